# Initial kernel scaffold; baseline (speedup 1.0000x reference)
import jax
import jax.numpy as jnp
from jax.experimental import pallas as pl


def kernel(x, edge_index, W1, b1, W2, b2):
    raise NotImplementedError("write your pallas kernel here")


# speedup vs baseline: 33.2670x; 33.2670x over previous
import functools

import jax
import jax.numpy as jnp
from jax import lax
from jax.experimental import pallas as pl
from jax.experimental.pallas import tpu as pltpu
from jax.experimental.pallas import tpu_sc as plsc

_NC = 2
_NS = 16
_NW = _NC * _NS
_LN = 128


def _sc_mesh():
    return plsc.VectorSubcoreMesh(
        core_axis_name="c", subcore_axis_name="s", num_cores=_NC, num_subcores=_NS
    )


def _make_deg_kernel(npad, ch):
    rpt = npad // _NS

    @functools.partial(
        pl.kernel,
        out_type=jax.ShapeDtypeStruct((_NC, npad), jnp.float32),
        mesh=_sc_mesh(),
        compiler_params=pltpu.CompilerParams(use_tc_tiling_on_sc=False),
        scratch_types=[
            pltpu.VMEM((ch, _LN), jnp.int32),
            pltpu.VMEM((_LN,), jnp.float32),
            pltpu.VMEM_SHARED((npad,), jnp.float32),
        ],
    )
    def deg_kernel(dstg_hbm, zeros_hbm, out_hbm, dst_v, ones_v, acc):
        cid = lax.axis_index("c")
        sid = lax.axis_index("s")
        wid = sid * _NC + cid
        pltpu.sync_copy(dstg_hbm.at[wid], dst_v)
        for i in range(_LN // 16):
            ones_v[pl.ds(i * 16, 16)] = jnp.full((16,), 1.0, jnp.float32)
        pltpu.sync_copy(zeros_hbm.at[pl.ds(sid * rpt, rpt)], acc.at[pl.ds(sid * rpt, rpt)])
        plsc.subcore_barrier()

        def body(j, carry):
            pltpu.sync_copy(ones_v, acc.at[dst_v.at[j]], add=True)
            return carry

        lax.fori_loop(0, ch, body, 0)
        plsc.subcore_barrier()
        pltpu.sync_copy(acc.at[pl.ds(sid * rpt, rpt)], out_hbm.at[cid, pl.ds(sid * rpt, rpt)])

    return deg_kernel


def _make_msg_kernel(npad, fh, ch):
    rpt = npad // _NS

    @functools.partial(
        pl.kernel,
        out_type=jax.ShapeDtypeStruct((_NC, npad, fh), jnp.float32),
        mesh=_sc_mesh(),
        compiler_params=pltpu.CompilerParams(use_tc_tiling_on_sc=False),
        scratch_types=[
            pltpu.VMEM((ch, _LN), jnp.int32),
            pltpu.VMEM((ch, _LN), jnp.int32),
            pltpu.VMEM((2, _LN, fh), jnp.float32),
            pltpu.VMEM_SHARED((npad, fh), jnp.float32),
            pltpu.SemaphoreType.DMA,
            pltpu.SemaphoreType.DMA,
        ],
    )
    def msg_kernel(y_hbm, srcg_hbm, dstg_hbm, zeros_hbm, out_hbm,
                   src_v, dst_v, buf, acc, sem0, sem1):
        cid = lax.axis_index("c")
        sid = lax.axis_index("s")
        wid = sid * _NC + cid
        pltpu.sync_copy(srcg_hbm.at[wid], src_v)
        pltpu.sync_copy(dstg_hbm.at[wid], dst_v)
        pltpu.sync_copy(zeros_hbm.at[pl.ds(sid * rpt, rpt)], acc.at[pl.ds(sid * rpt, rpt)])
        plsc.subcore_barrier()

        def body(g, carry):
            j0 = 2 * g
            j1 = 2 * g + 1
            c0 = pltpu.async_copy(y_hbm.at[src_v.at[j0]], buf.at[0], sem0)
            c1 = pltpu.async_copy(y_hbm.at[src_v.at[j1]], buf.at[1], sem1)
            c0.wait()
            pltpu.sync_copy(buf.at[0], acc.at[dst_v.at[j0]], add=True)
            c1.wait()
            pltpu.sync_copy(buf.at[1], acc.at[dst_v.at[j1]], add=True)
            return carry

        lax.fori_loop(0, ch // 2, body, 0)
        plsc.subcore_barrier()
        pltpu.sync_copy(acc.at[pl.ds(sid * rpt, rpt)], out_hbm.at[cid, pl.ds(sid * rpt, rpt)])

    return msg_kernel


def _tc1_body(x_ref, w1_ref, d0_ref, d1_ref, dinv_ref, y_ref):
    deg = 1.0 + d0_ref[...] + d1_ref[...]
    dinv = lax.rsqrt(deg)
    dinv_ref[...] = dinv
    xw = jnp.dot(x_ref[...], w1_ref[...], preferred_element_type=jnp.float32)
    y_ref[...] = xw * dinv


def _tc2_body(y1_ref, s_ref, dinv_ref, b1_ref, w2_ref, y2_ref, *, n):
    s = s_ref[...]
    tot = y1_ref[...] + s[0] + s[1]
    dinv = dinv_ref[...]
    h = jnp.maximum(tot * dinv + b1_ref[...], 0.0)
    rows = lax.broadcasted_iota(jnp.int32, h.shape, 0)
    h = jnp.where(rows < n, h, 0.0)
    y2_ref[...] = jnp.dot(h, w2_ref[...], preferred_element_type=jnp.float32) * dinv


def _tc3_body(y2_ref, s_ref, dinv_ref, b2_ref, out_ref, *, n):
    s = s_ref[...]
    tot = (y2_ref[...] + s[0] + s[1]) * dinv_ref[...] + b2_ref[...]
    out_ref[...] = tot[:n, :]


def kernel(x, edge_index, W1, b1, W2, b2):
    n, fin = x.shape
    fh = W1.shape[1]
    e = edge_index.shape[1]
    f32 = jnp.float32

    npad = ((n + 16 + 511) // 512) * 512
    ch = -(-e // (_NW * _LN))
    if ch % 2:
        ch += 1
    epad = _NW * _LN * ch

    src = edge_index[0].astype(jnp.int32)
    dst = edge_index[1].astype(jnp.int32)
    pad_idx = n + (jnp.arange(epad - e, dtype=jnp.int32) % (npad - n))
    srcg = jnp.concatenate([src, pad_idx]).reshape(_NW, ch, _LN)
    dstg = jnp.concatenate([dst, pad_idx]).reshape(_NW, ch, _LN)

    zeros1 = jnp.zeros((npad,), f32)
    zeros2 = jnp.zeros((npad, fh), f32)
    x_pad = jnp.concatenate([x, jnp.zeros((npad - n, fin), x.dtype)], axis=0)

    deg_parts = _make_deg_kernel(npad, ch)(dstg, zeros1)
    d0 = deg_parts[0].reshape(npad, 1)
    d1 = deg_parts[1].reshape(npad, 1)

    dinv, y1 = pl.pallas_call(
        _tc1_body,
        out_shape=[
            jax.ShapeDtypeStruct((npad, 1), f32),
            jax.ShapeDtypeStruct((npad, fh), f32),
        ],
    )(x_pad, W1, d0, d1)

    msg = _make_msg_kernel(npad, fh, ch)
    s1 = msg(y1, srcg, dstg, zeros2)

    y2 = pl.pallas_call(
        functools.partial(_tc2_body, n=n),
        out_shape=jax.ShapeDtypeStruct((npad, fh), f32),
    )(y1, s1, dinv, b1.reshape(1, fh), W2)

    s2 = msg(y2, srcg, dstg, zeros2)

    out = pl.pallas_call(
        functools.partial(_tc3_body, n=n),
        out_shape=jax.ShapeDtypeStruct((n, fh), f32),
    )(y2, s2, dinv, b2.reshape(1, fh))
    return out


# speedup vs baseline: 41.7235x; 1.2542x over previous
import functools

import jax
import jax.numpy as jnp
from jax import lax
from jax.experimental import pallas as pl
from jax.experimental.pallas import tpu as pltpu
from jax.experimental.pallas import tpu_sc as plsc

_NC = 2
_NS = 16
_NW = _NC * _NS
_LN = 128


def _sc_mesh():
    return plsc.VectorSubcoreMesh(
        core_axis_name="c", subcore_axis_name="s", num_cores=_NC, num_subcores=_NS
    )


def _make_deg_kernel(npad, ch):
    rpt = npad // _NS

    @functools.partial(
        pl.kernel,
        out_type=jax.ShapeDtypeStruct((_NC, npad), jnp.float32),
        mesh=_sc_mesh(),
        compiler_params=pltpu.CompilerParams(use_tc_tiling_on_sc=False),
        scratch_types=[
            pltpu.VMEM((ch, _LN), jnp.int32),
            pltpu.VMEM((_LN,), jnp.float32),
            pltpu.VMEM_SHARED((npad,), jnp.float32),
            pltpu.SemaphoreType.DMA,
        ],
    )
    def deg_kernel(dstg_hbm, zeros_hbm, out_hbm, dst_v, ones_v, acc, ssem):
        cid = lax.axis_index("c")
        sid = lax.axis_index("s")
        wid = sid * _NC + cid
        pltpu.sync_copy(dstg_hbm.at[wid], dst_v)
        for i in range(_LN // 16):
            ones_v[pl.ds(i * 16, 16)] = jnp.full((16,), 1.0, jnp.float32)
        pltpu.sync_copy(zeros_hbm.at[pl.ds(sid * rpt, rpt)], acc.at[pl.ds(sid * rpt, rpt)])
        plsc.subcore_barrier()

        def body(g, carry):
            for i in range(8):
                pltpu.async_copy(ones_v, acc.at[dst_v.at[8 * g + i]], ssem, add=True)
            for i in range(8):
                pltpu.make_async_copy(ones_v, acc.at[dst_v.at[8 * g + i]], ssem).wait()
            return carry

        lax.fori_loop(0, ch // 8, body, 0)
        plsc.subcore_barrier()
        pltpu.sync_copy(acc.at[pl.ds(sid * rpt, rpt)], out_hbm.at[cid, pl.ds(sid * rpt, rpt)])

    return deg_kernel


def _make_msg_kernel(npad, fh, ch):
    rpt = npad // _NS

    @functools.partial(
        pl.kernel,
        out_type=jax.ShapeDtypeStruct((_NC, npad, fh), jnp.float32),
        mesh=_sc_mesh(),
        compiler_params=pltpu.CompilerParams(use_tc_tiling_on_sc=False),
        scratch_types=[
            pltpu.VMEM((ch, _LN), jnp.int32),
            pltpu.VMEM((ch, _LN), jnp.int32),
            pltpu.VMEM((8, _LN, fh), jnp.float32),
            pltpu.VMEM_SHARED((npad, fh), jnp.float32),
            pltpu.SemaphoreType.DMA,
            pltpu.SemaphoreType.DMA,
        ],
    )
    def msg_kernel(y_hbm, srcg_hbm, dstg_hbm, zeros_hbm, out_hbm,
                   src_v, dst_v, buf, acc, gsem, ssem):
        cid = lax.axis_index("c")
        sid = lax.axis_index("s")
        wid = sid * _NC + cid
        pltpu.sync_copy(srcg_hbm.at[wid], src_v)
        pltpu.sync_copy(dstg_hbm.at[wid], dst_v)
        pltpu.sync_copy(zeros_hbm.at[pl.ds(sid * rpt, rpt)], acc.at[pl.ds(sid * rpt, rpt)])
        plsc.subcore_barrier()

        ngrp = ch // 4
        nit = ngrp // 2

        def fire_g(half, grp):
            for i in range(4):
                pltpu.async_copy(y_hbm.at[src_v.at[4 * grp + i]], buf.at[4 * half + i], gsem)

        def drain_g(half, grp):
            for i in range(4):
                pltpu.make_async_copy(y_hbm.at[src_v.at[4 * grp + i]], buf.at[4 * half + i], gsem).wait()

        def fire_s(half, grp):
            for i in range(4):
                pltpu.async_copy(buf.at[4 * half + i], acc.at[dst_v.at[4 * grp + i]], ssem, add=True)

        def drain_s(half, grp):
            for i in range(4):
                pltpu.make_async_copy(buf.at[4 * half + i], acc.at[dst_v.at[4 * grp + i]], ssem).wait()

        fire_g(0, 0)

        def body(gg, carry):
            g0 = 2 * gg
            g1 = 2 * gg + 1
            drain_g(0, g0)
            fire_g(1, g1)
            fire_s(0, g0)
            drain_s(0, g0)
            drain_g(1, g1)

            @pl.when(gg < nit - 1)
            def _():
                fire_g(0, g0 + 2)

            fire_s(1, g1)
            drain_s(1, g1)
            return carry

        lax.fori_loop(0, nit, body, 0)
        plsc.subcore_barrier()
        pltpu.sync_copy(acc.at[pl.ds(sid * rpt, rpt)], out_hbm.at[cid, pl.ds(sid * rpt, rpt)])

    return msg_kernel


def _tc1_body(x_ref, w1_ref, d0_ref, d1_ref, dinv_ref, y_ref):
    deg = 1.0 + d0_ref[...] + d1_ref[...]
    dinv = lax.rsqrt(deg)
    dinv_ref[...] = dinv
    xw = jnp.dot(x_ref[...], w1_ref[...], preferred_element_type=jnp.float32)
    y_ref[...] = xw * dinv


def _tc2_body(y1_ref, s_ref, dinv_ref, b1_ref, w2_ref, y2_ref, *, n):
    s = s_ref[...]
    tot = y1_ref[...] + s[0] + s[1]
    dinv = dinv_ref[...]
    h = jnp.maximum(tot * dinv + b1_ref[...], 0.0)
    rows = lax.broadcasted_iota(jnp.int32, h.shape, 0)
    h = jnp.where(rows < n, h, 0.0)
    y2_ref[...] = jnp.dot(h, w2_ref[...], preferred_element_type=jnp.float32) * dinv


def _tc3_body(y2_ref, s_ref, dinv_ref, b2_ref, out_ref, *, n):
    s = s_ref[...]
    tot = (y2_ref[...] + s[0] + s[1]) * dinv_ref[...] + b2_ref[...]
    out_ref[...] = tot[:n, :]


def kernel(x, edge_index, W1, b1, W2, b2):
    n, fin = x.shape
    fh = W1.shape[1]
    e = edge_index.shape[1]
    f32 = jnp.float32

    npad = ((n + 16 + 511) // 512) * 512
    ch = -(-e // (_NW * _LN))
    ch = ((ch + 7) // 8) * 8
    epad = _NW * _LN * ch

    src = edge_index[0].astype(jnp.int32)
    dst = edge_index[1].astype(jnp.int32)
    pad_idx = n + (jnp.arange(epad - e, dtype=jnp.int32) % (npad - n))
    srcg = jnp.concatenate([src, pad_idx]).reshape(_NW, ch, _LN)
    dstg = jnp.concatenate([dst, pad_idx]).reshape(_NW, ch, _LN)

    zeros1 = jnp.zeros((npad,), f32)
    zeros2 = jnp.zeros((npad, fh), f32)
    x_pad = jnp.concatenate([x, jnp.zeros((npad - n, fin), x.dtype)], axis=0)

    deg_parts = _make_deg_kernel(npad, ch)(dstg, zeros1)
    d0 = deg_parts[0].reshape(npad, 1)
    d1 = deg_parts[1].reshape(npad, 1)

    dinv, y1 = pl.pallas_call(
        _tc1_body,
        out_shape=[
            jax.ShapeDtypeStruct((npad, 1), f32),
            jax.ShapeDtypeStruct((npad, fh), f32),
        ],
    )(x_pad, W1, d0, d1)

    msg = _make_msg_kernel(npad, fh, ch)
    s1 = msg(y1, srcg, dstg, zeros2)

    y2 = pl.pallas_call(
        functools.partial(_tc2_body, n=n),
        out_shape=jax.ShapeDtypeStruct((npad, fh), f32),
    )(y1, s1, dinv, b1.reshape(1, fh), W2)

    s2 = msg(y2, srcg, dstg, zeros2)

    out = pl.pallas_call(
        functools.partial(_tc3_body, n=n),
        out_shape=jax.ShapeDtypeStruct((n, fh), f32),
    )(y2, s2, dinv, b2.reshape(1, fh))
    return out
